# baseline (speedup 1.0000x reference)
import functools

import jax
import jax.numpy as jnp
from jax import lax
from jax.experimental import pallas as pl
from jax.experimental.pallas import tpu as pltpu
from jax.experimental.pallas import tpu_sc as plsc

VOCAB = 10000
EMBED_DIM = 64
BATCH = 16384
BAG_LEN = 50

NUM_CORES = 2
NUM_SUBCORES = 16
LANES = 16
NUM_WORKERS = NUM_CORES * NUM_SUBCORES
BAGS_PER_W = BATCH // NUM_WORKERS
GROUPS = BAGS_PER_W // LANES

SCHUNK = 640
VOCAB_PAD = SCHUNK * NUM_SUBCORES
TAIL0 = 9472


@functools.partial(
    pl.kernel,
    mesh=plsc.VectorSubcoreMesh(core_axis_name="c", subcore_axis_name="s"),
    out_type=jax.ShapeDtypeStruct((BATCH,), jnp.float32),
    compiler_params=pltpu.CompilerParams(needs_layout_passes=False),
    scratch_types=[
        pltpu.VMEM((EMBED_DIM, SCHUNK), jnp.float32),
        pltpu.VMEM((EMBED_DIM, EMBED_DIM), jnp.float32),
        pltpu.VMEM((EMBED_DIM,), jnp.float32),
        pltpu.VMEM((EMBED_DIM,), jnp.float32),
        pltpu.VMEM((16,), jnp.float32),
        pltpu.VMEM((SCHUNK,), jnp.float32),
        pltpu.VMEM((VOCAB_PAD,), jnp.float32),
        pltpu.VMEM((BAG_LEN, BAGS_PER_W), jnp.int32),
        pltpu.VMEM((BAGS_PER_W,), jnp.float32),
        pltpu.VMEM_SHARED((VOCAB_PAD,), jnp.float32),
        pltpu.SemaphoreType.DMA,
        pltpu.SemaphoreType.DMA,
    ],
)
def _sc_embed_nn(tt_hbm, w1_hbm, w2_hbm, b1_hbm, b2_hbm, idxt_hbm, out_hbm,
                 tt_v, w1_v, w2_v, b1_v, b2_v, sch_v, s_v, idx_v, out_v,
                 s_sh, sem_t, sem_i):
    cid = lax.axis_index("c")
    sid = lax.axis_index("s")
    wid = sid * NUM_CORES + cid
    bag0 = wid * BAGS_PER_W
    col0 = pl.multiple_of(
        jnp.where(sid == NUM_SUBCORES - 1, TAIL0, sid * SCHUNK), 128)

    cp_i = pltpu.async_copy(
        idxt_hbm.at[:, pl.ds(bag0, BAGS_PER_W)], idx_v, sem_i)
    pltpu.async_copy(tt_hbm.at[:, pl.ds(col0, SCHUNK)], tt_v, sem_t).wait()

    pltpu.sync_copy(w1_hbm, w1_v)
    pltpu.sync_copy(w2_hbm, w2_v)
    pltpu.sync_copy(b1_hbm, b1_v)
    pltpu.sync_copy(b2_hbm, b2_v)

    w2c = [w2_v[pl.ds(16 * q, 16)] for q in range(4)]
    vc = []
    for q in range(4):
        acc = jnp.zeros((16,), jnp.float32)
        for k in range(EMBED_DIM):
            acc = acc + w2c[k // 16][k % 16] * w1_v[k, pl.ds(16 * q, 16)]
        vc.append(acc)
    cacc = jnp.zeros((16,), jnp.float32)
    for q in range(4):
        cacc = cacc + b1_v[pl.ds(16 * q, 16)] * w2c[q]
    c = lax.reduce_sum_p.bind(cacc, axes=(0,)) + b2_v[pl.ds(0, 16)][0]

    def fold_body(j, carry):
        col = j * LANES
        acc = jnp.zeros((16,), jnp.float32)
        for d in range(EMBED_DIM):
            acc = acc + vc[d // 16][d % 16] * tt_v[d, pl.ds(col, LANES)]
        sch_v[pl.ds(col, LANES)] = (acc + c) * (1.0 / BAG_LEN)
        return carry

    lax.fori_loop(0, SCHUNK // LANES, fold_body, 0)

    pltpu.sync_copy(sch_v, s_sh.at[pl.ds(col0, SCHUNK)])
    plsc.subcore_barrier()
    pltpu.sync_copy(s_sh, s_v)
    cp_i.wait()

    @plsc.parallel_loop(0, GROUPS, 1)
    def group_body(j):
        col = j * LANES
        accs = [jnp.zeros((16,), jnp.float32) for _ in range(4)]
        for l in range(BAG_LEN):
            iv = idx_v[l, pl.ds(col, LANES)]
            accs[l % 4] = accs[l % 4] + plsc.load_gather(s_v, [iv])
        out_v[pl.ds(col, LANES)] = (accs[0] + accs[1]) + (accs[2] + accs[3])

    pltpu.sync_copy(out_v, out_hbm.at[pl.ds(bag0, BAGS_PER_W)])


def kernel(input, emb_table, W1, b1, W2, b2):
    idx_t = input.astype(jnp.int32).T
    out = _sc_embed_nn(
        emb_table.T, W1, W2.reshape(EMBED_DIM), b1,
        jnp.pad(b2, (0, 15)), idx_t)
    return out.reshape(BATCH, 1)


# speedup vs baseline: 1.0110x; 1.0110x over previous
import functools

import jax
import jax.numpy as jnp
from jax import lax
from jax.experimental import pallas as pl
from jax.experimental.pallas import tpu as pltpu
from jax.experimental.pallas import tpu_sc as plsc

VOCAB = 10000
EMBED_DIM = 64
BATCH = 16384
BAG_LEN = 50

NUM_CORES = 2
NUM_SUBCORES = 16
LANES = 16
NUM_WORKERS = NUM_CORES * NUM_SUBCORES
BAGS_PER_W = BATCH // NUM_WORKERS
GROUPS = BAGS_PER_W // LANES

SCHUNK = 640
VOCAB_PAD = SCHUNK * NUM_SUBCORES
TAIL0 = 9472


@functools.partial(
    pl.kernel,
    mesh=plsc.VectorSubcoreMesh(core_axis_name="c", subcore_axis_name="s"),
    out_type=jax.ShapeDtypeStruct((BATCH,), jnp.float32),
    compiler_params=pltpu.CompilerParams(needs_layout_passes=False),
    scratch_types=[
        pltpu.VMEM((EMBED_DIM, SCHUNK), jnp.float32),
        pltpu.VMEM((EMBED_DIM, EMBED_DIM), jnp.float32),
        pltpu.VMEM((EMBED_DIM,), jnp.float32),
        pltpu.VMEM((EMBED_DIM,), jnp.float32),
        pltpu.VMEM((16,), jnp.float32),
        pltpu.VMEM((SCHUNK,), jnp.float32),
        pltpu.VMEM((VOCAB_PAD,), jnp.float32),
        pltpu.VMEM((BAG_LEN, BAGS_PER_W), jnp.int32),
        pltpu.VMEM((BAGS_PER_W,), jnp.float32),
        pltpu.VMEM_SHARED((VOCAB_PAD,), jnp.float32),
        pltpu.SemaphoreType.DMA,
        pltpu.SemaphoreType.DMA,
    ],
)
def _sc_embed_nn(tt_hbm, w1_hbm, w2_hbm, b1_hbm, b2_hbm, idxt_hbm, out_hbm,
                 tt_v, w1_v, w2_v, b1_v, b2_v, sch_v, s_v, idx_v, out_v,
                 s_sh, sem_t, sem_i):
    cid = lax.axis_index("c")
    sid = lax.axis_index("s")
    wid = sid * NUM_CORES + cid
    bag0 = wid * BAGS_PER_W
    col0 = pl.multiple_of(
        jnp.where(sid == NUM_SUBCORES - 1, TAIL0, sid * SCHUNK), 128)

    cp_i = pltpu.async_copy(
        idxt_hbm.at[:, pl.ds(bag0, BAGS_PER_W)], idx_v, sem_i)
    pltpu.async_copy(tt_hbm.at[:, pl.ds(col0, SCHUNK)], tt_v, sem_t).wait()

    pltpu.sync_copy(w1_hbm, w1_v)
    pltpu.sync_copy(w2_hbm, w2_v)
    pltpu.sync_copy(b1_hbm, b1_v)
    pltpu.sync_copy(b2_hbm, b2_v)

    w2c = [w2_v[pl.ds(16 * q, 16)] for q in range(4)]
    vc = []
    for q in range(4):
        acc = jnp.zeros((16,), jnp.float32)
        for k in range(EMBED_DIM):
            acc = acc + w2c[k // 16][k % 16] * w1_v[k, pl.ds(16 * q, 16)]
        vc.append(acc)
    cacc = jnp.zeros((16,), jnp.float32)
    for q in range(4):
        cacc = cacc + b1_v[pl.ds(16 * q, 16)] * w2c[q]
    c = lax.reduce_sum_p.bind(cacc, axes=(0,)) + b2_v[pl.ds(0, 16)][0]

    def fold_body(j, carry):
        col = j * LANES
        accs = [jnp.zeros((16,), jnp.float32) for _ in range(4)]
        for d in range(EMBED_DIM):
            accs[d % 4] = accs[d % 4] + (
                vc[d // 16][d % 16] * tt_v[d, pl.ds(col, LANES)])
        sch_v[pl.ds(col, LANES)] = (
            (accs[0] + accs[1]) + (accs[2] + accs[3]) + c) * (1.0 / BAG_LEN)
        return carry

    lax.fori_loop(0, SCHUNK // LANES, fold_body, 0)

    pltpu.sync_copy(sch_v, s_sh.at[pl.ds(col0, SCHUNK)])
    plsc.subcore_barrier()
    pltpu.sync_copy(s_sh, s_v)
    cp_i.wait()

    @plsc.parallel_loop(0, GROUPS, 1)
    def group_body(j):
        col = j * LANES
        accs = [jnp.zeros((16,), jnp.float32) for _ in range(4)]
        for l in range(BAG_LEN):
            iv = idx_v[l, pl.ds(col, LANES)]
            accs[l % 4] = accs[l % 4] + plsc.load_gather(s_v, [iv])
        out_v[pl.ds(col, LANES)] = (accs[0] + accs[1]) + (accs[2] + accs[3])

    pltpu.sync_copy(out_v, out_hbm.at[pl.ds(bag0, BAGS_PER_W)])


def kernel(input, emb_table, W1, b1, W2, b2):
    idx_t = input.astype(jnp.int32).T
    out = _sc_embed_nn(
        emb_table.T, W1, W2.reshape(EMBED_DIM), b1,
        jnp.pad(b2, (0, 15)), idx_t)
    return out.reshape(BATCH, 1)
